# baseline (speedup 1.0000x reference)
import functools

import jax
import jax.numpy as jnp
from jax import lax
from jax.experimental import pallas as pl
from jax.experimental.pallas import tpu as pltpu
from jax.experimental.pallas import tpu_sc as plsc

N_NODES = 10000
OUT_C = 64
ALPHA = 0.1
K_STEPS = 10
BETA = 1.0 - ALPHA

NCORES = 2
NSUB = 16
NTILES = NCORES * NSUB
CHUNK = 128
NBUF = 8
AHEAD = 4

ROW_BLK = 400


def _make_edge_kernel(n_chunks, per_tile, n_acc):
    rows_acc = n_acc // NSUB

    mesh = plsc.VectorSubcoreMesh(core_axis_name="c", subcore_axis_name="s")

    @functools.partial(
        pl.kernel,
        mesh=mesh,
        compiler_params=pltpu.CompilerParams(use_tc_tiling_on_sc=False),
        out_type=jax.ShapeDtypeStruct((NCORES, n_acc, OUT_C), jnp.float32),
        scratch_types=[
            pltpu.VMEM((n_chunks, CHUNK), jnp.int32),
            pltpu.VMEM((n_chunks, CHUNK), jnp.int32),
            [pltpu.VMEM((CHUNK, OUT_C), jnp.float32)] * NBUF,
            pltpu.VMEM_SHARED((n_acc, OUT_C), jnp.float32),
            [pltpu.SemaphoreType.DMA] * NBUF,
            [pltpu.SemaphoreType.DMA] * NBUF,
        ],
    )
    def edge_kernel(src_hbm, dst_hbm, zs_hbm, zero_hbm, out_hbm,
                    srcv, dstv, rows, acc, gsem, ssem):
        c = lax.axis_index("c")
        s = lax.axis_index("s")
        wid = c * NSUB + s
        pltpu.sync_copy(src_hbm.at[pl.ds(wid * n_chunks, n_chunks)], srcv)
        pltpu.sync_copy(dst_hbm.at[pl.ds(wid * n_chunks, n_chunks)], dstv)
        pltpu.sync_copy(zero_hbm, acc.at[pl.ds(s * rows_acc, rows_acc)])
        plsc.subcore_barrier()

        for b in range(AHEAD):
            pltpu.async_copy(zs_hbm.at[srcv.at[b]], rows[b], gsem[b])

        def ring_body(g, carry):
            for i in range(NBUF):
                k = NBUF * g + i
                b2 = (i + AHEAD) % NBUF
                pltpu.make_async_copy(
                    zs_hbm.at[srcv.at[k]], rows[i], gsem[i]).wait()
                pltpu.async_copy(rows[i], acc.at[dstv.at[k]], ssem[i],
                                 add=True)
                j = k + AHEAD

                @pl.when(jnp.logical_and(k >= AHEAD, j < n_chunks))
                def _():
                    pltpu.make_async_copy(
                        zs_hbm.at[pl.ds(0, CHUNK)], rows[b2],
                        ssem[b2]).wait()

                @pl.when(j < n_chunks)
                def _():
                    pltpu.async_copy(zs_hbm.at[srcv.at[j]], rows[b2],
                                     gsem[b2])

            return carry

        lax.fori_loop(0, n_chunks // NBUF, ring_body, 0)
        for b in range(NBUF):
            pltpu.make_async_copy(zs_hbm.at[pl.ds(0, CHUNK)], rows[b],
                                  ssem[b]).wait()
        plsc.subcore_barrier()
        pltpu.sync_copy(acc.at[pl.ds(s * rows_acc, rows_acc)],
                        out_hbm.at[c, pl.ds(s * rows_acc, rows_acc)])

    return edge_kernel


def _mlp_body(x_ref, w1_ref, b1_ref, w2_ref, b2_ref, h_ref, g_ref):
    hh = jnp.dot(x_ref[...], w1_ref[...], preferred_element_type=jnp.float32)
    hh = jnp.maximum(hh + b1_ref[...], 0.0)
    out = jnp.dot(hh, w2_ref[...], preferred_element_type=jnp.float32)
    out = out + b2_ref[...]
    h_ref[...] = out
    g_ref[...] = ALPHA * out


def _deg_dinv(d0_ref, d1_ref):
    deg = d0_ref[:, 0:1] + d1_ref[:, 0:1] + 1.0
    return deg, lax.rsqrt(deg)


def _prep2_body(h_ref, d0_ref, d1_ref, zs_ref):
    _, dinv = _deg_dinv(d0_ref, d1_ref)
    zs_ref[...] = dinv * h_ref[...]


def _combine_math(p0_ref, p1_ref, z_ref, g_ref, d0_ref, d1_ref):
    deg, dinv = _deg_dinv(d0_ref, d1_ref)
    s = p0_ref[...] + p1_ref[...]
    zn = (BETA * dinv) * s + (BETA / deg) * z_ref[...] + g_ref[...]
    return zn, dinv


def _combine_body(p0_ref, p1_ref, z_ref, g_ref, d0_ref, d1_ref,
                  zn_ref, zs_ref):
    zn, dinv = _combine_math(p0_ref, p1_ref, z_ref, g_ref, d0_ref, d1_ref)
    zn_ref[...] = zn
    zs_ref[...] = dinv * zn


def _final_body(p0_ref, p1_ref, z_ref, g_ref, d0_ref, d1_ref, out_ref):
    zn, _ = _combine_math(p0_ref, p1_ref, z_ref, g_ref, d0_ref, d1_ref)
    m = jnp.max(zn, axis=1, keepdims=True)
    e = jnp.exp(zn - m)
    lse = jnp.log(jnp.sum(e, axis=1, keepdims=True))
    out_ref[...] = zn - m - lse


def _row_spec(cols):
    return pl.BlockSpec((ROW_BLK, cols), lambda i: (i, 0))


def _full_spec(r, c):
    return pl.BlockSpec((r, c), lambda i: (0, 0))


_GRID = (N_NODES // ROW_BLK,)
_F32 = jnp.float32


def _sds(shape):
    return jax.ShapeDtypeStruct(shape, _F32)


def kernel(x, edge_index, W1, b1, W2, b2):
    src = edge_index[0].astype(jnp.int32)
    dst = edge_index[1].astype(jnp.int32)
    e_cnt = src.shape[0]
    per_tile = -(-e_cnt // (NTILES * NBUF * CHUNK)) * NBUF * CHUNK
    n_chunks = per_tile // CHUNK
    e_pad = NTILES * per_tile
    n_acc = -(-(N_NODES + 1) // (NSUB * 8)) * (NSUB * 8)
    pad = e_pad - e_cnt
    src_p = jnp.concatenate([src, jnp.zeros((pad,), jnp.int32)])
    dst_p = jnp.concatenate([dst, jnp.full((pad,), N_NODES, jnp.int32)])
    src_p = src_p.reshape(NTILES * n_chunks, CHUNK)
    dst_p = dst_p.reshape(NTILES * n_chunks, CHUNK)
    zero_blk = jnp.zeros((n_acc // NSUB, OUT_C), _F32)
    ones_tab = jnp.ones((N_NODES, OUT_C), _F32)

    edge_call = _make_edge_kernel(n_chunks, per_tile, n_acc)

    h, g = pl.pallas_call(
        _mlp_body,
        grid=_GRID,
        in_specs=[_row_spec(128), _full_spec(128, 128), _full_spec(1, 128),
                  _full_spec(128, OUT_C), _full_spec(1, OUT_C)],
        out_specs=[_row_spec(OUT_C)] * 2,
        out_shape=[_sds((N_NODES, OUT_C))] * 2,
    )(x, W1, b1.reshape(1, -1), W2, b2.reshape(1, -1))

    deg_p = edge_call(src_p, dst_p, ones_tab, zero_blk)
    d0, d1 = deg_p[0, :N_NODES], deg_p[1, :N_NODES]

    zs = pl.pallas_call(
        _prep2_body,
        grid=_GRID,
        in_specs=[_row_spec(OUT_C)] * 3,
        out_specs=_row_spec(OUT_C),
        out_shape=_sds((N_NODES, OUT_C)),
    )(h, d0, d1)

    z = h
    for r in range(K_STEPS):
        p_full = edge_call(src_p, dst_p, zs, zero_blk)
        p = p_full[:, :N_NODES]
        if r < K_STEPS - 1:
            z, zs = pl.pallas_call(
                _combine_body,
                grid=_GRID,
                in_specs=[_row_spec(OUT_C)] * 6,
                out_specs=[_row_spec(OUT_C)] * 2,
                out_shape=[_sds((N_NODES, OUT_C))] * 2,
            )(p[0], p[1], z, g, d0, d1)
        else:
            out = pl.pallas_call(
                _final_body,
                grid=_GRID,
                in_specs=[_row_spec(OUT_C)] * 6,
                out_specs=_row_spec(OUT_C),
                out_shape=_sds((N_NODES, OUT_C)),
            )(p[0], p[1], z, g, d0, d1)
    return out


# speedup vs baseline: 1.1675x; 1.1675x over previous
import functools

import jax
import jax.numpy as jnp
from jax import lax
from jax.experimental import pallas as pl
from jax.experimental.pallas import tpu as pltpu
from jax.experimental.pallas import tpu_sc as plsc

N_NODES = 10000
OUT_C = 64
ALPHA = 0.1
K_STEPS = 10
BETA = 1.0 - ALPHA

NCORES = 2
NSUB = 16
NTILES = NCORES * NSUB
CHUNK = 128
DEG_W = 16

ROW_BLK = 400


def _make_edge_kernel(n_chunks, n_acc, width):
    rows_acc = n_acc // NSUB

    mesh = plsc.VectorSubcoreMesh(core_axis_name="c", subcore_axis_name="s")

    @functools.partial(
        pl.kernel,
        mesh=mesh,
        compiler_params=pltpu.CompilerParams(use_tc_tiling_on_sc=False),
        out_type=jax.ShapeDtypeStruct((NCORES, n_acc, width), jnp.float32),
        scratch_types=[
            pltpu.VMEM((n_chunks, CHUNK), jnp.int32),
            pltpu.VMEM((n_chunks, CHUNK), jnp.int32),
            pltpu.VMEM((CHUNK, width), jnp.float32),
            pltpu.VMEM((CHUNK, width), jnp.float32),
            pltpu.VMEM_SHARED((n_acc, width), jnp.float32),
            pltpu.SemaphoreType.DMA,
            pltpu.SemaphoreType.DMA,
        ],
    )
    def edge_kernel(src_hbm, dst_hbm, zs_hbm, zero_hbm, out_hbm,
                    srcv, dstv, rows0, rows1, acc, sem0, sem1):
        c = lax.axis_index("c")
        s = lax.axis_index("s")
        wid = c * NSUB + s
        pltpu.sync_copy(src_hbm.at[pl.ds(wid * n_chunks, n_chunks)], srcv)
        pltpu.sync_copy(dst_hbm.at[pl.ds(wid * n_chunks, n_chunks)], dstv)
        pltpu.sync_copy(zero_hbm, acc.at[pl.ds(s * rows_acc, rows_acc)])
        plsc.subcore_barrier()

        pltpu.async_copy(zs_hbm.at[srcv.at[0]], rows0, sem0)
        pltpu.async_copy(zs_hbm.at[srcv.at[1]], rows1, sem1)

        def pair_body(g, carry):
            for b, rb, sb in ((0, rows0, sem0), (1, rows1, sem1)):
                k = 2 * g + b
                pltpu.make_async_copy(zs_hbm.at[srcv.at[k]], rb, sb).wait()
                pltpu.sync_copy(rb, acc.at[dstv.at[k]], add=True)

                @pl.when(k + 2 < n_chunks)
                def _():
                    pltpu.async_copy(zs_hbm.at[srcv.at[k + 2]], rb, sb)

            return carry

        lax.fori_loop(0, n_chunks // 2, pair_body, 0)
        plsc.subcore_barrier()
        pltpu.sync_copy(acc.at[pl.ds(s * rows_acc, rows_acc)],
                        out_hbm.at[c, pl.ds(s * rows_acc, rows_acc)])

    return edge_kernel


def _mlp_body(x_ref, w1_ref, b1_ref, w2_ref, b2_ref, h_ref, g_ref):
    hh = jnp.dot(x_ref[...], w1_ref[...], preferred_element_type=jnp.float32)
    hh = jnp.maximum(hh + b1_ref[...], 0.0)
    out = jnp.dot(hh, w2_ref[...], preferred_element_type=jnp.float32)
    out = out + b2_ref[...]
    h_ref[...] = out
    g_ref[...] = ALPHA * out


def _deg_dinv(d0_ref, d1_ref):
    deg = d0_ref[:, 0:1] + d1_ref[:, 0:1] + 1.0
    return deg, lax.rsqrt(deg)


def _prep2_body(h_ref, d0_ref, d1_ref, zs_ref):
    _, dinv = _deg_dinv(d0_ref, d1_ref)
    zs_ref[...] = dinv * h_ref[...]


def _combine_math(p0_ref, p1_ref, z_ref, g_ref, d0_ref, d1_ref):
    deg, dinv = _deg_dinv(d0_ref, d1_ref)
    s = p0_ref[...] + p1_ref[...]
    zn = (BETA * dinv) * s + (BETA / deg) * z_ref[...] + g_ref[...]
    return zn, dinv


def _combine_body(p0_ref, p1_ref, z_ref, g_ref, d0_ref, d1_ref,
                  zn_ref, zs_ref):
    zn, dinv = _combine_math(p0_ref, p1_ref, z_ref, g_ref, d0_ref, d1_ref)
    zn_ref[...] = zn
    zs_ref[...] = dinv * zn


def _final_body(p0_ref, p1_ref, z_ref, g_ref, d0_ref, d1_ref, out_ref):
    zn, _ = _combine_math(p0_ref, p1_ref, z_ref, g_ref, d0_ref, d1_ref)
    m = jnp.max(zn, axis=1, keepdims=True)
    e = jnp.exp(zn - m)
    lse = jnp.log(jnp.sum(e, axis=1, keepdims=True))
    out_ref[...] = zn - m - lse


def _row_spec(cols):
    return pl.BlockSpec((ROW_BLK, cols), lambda i: (i, 0))


def _full_spec(r, c):
    return pl.BlockSpec((r, c), lambda i: (0, 0))


_GRID = (N_NODES // ROW_BLK,)
_F32 = jnp.float32


def _sds(shape):
    return jax.ShapeDtypeStruct(shape, _F32)


def kernel(x, edge_index, W1, b1, W2, b2):
    src = edge_index[0].astype(jnp.int32)
    dst = edge_index[1].astype(jnp.int32)
    e_cnt = src.shape[0]
    per_tile = -(-e_cnt // (NTILES * 2 * CHUNK)) * 2 * CHUNK
    n_chunks = per_tile // CHUNK
    e_pad = NTILES * per_tile
    n_acc = -(-(N_NODES + 1) // (NSUB * 8)) * (NSUB * 8)
    pad = e_pad - e_cnt
    src_p = jnp.concatenate([src, jnp.zeros((pad,), jnp.int32)])
    dst_p = jnp.concatenate([dst, jnp.full((pad,), N_NODES, jnp.int32)])
    src_p = src_p.reshape(NTILES * n_chunks, CHUNK)
    dst_p = dst_p.reshape(NTILES * n_chunks, CHUNK)
    zero_blk = jnp.zeros((n_acc // NSUB, OUT_C), _F32)
    zero_blk_deg = jnp.zeros((n_acc // NSUB, DEG_W), _F32)
    ones_tab = jnp.ones((N_NODES, DEG_W), _F32)

    edge_call = _make_edge_kernel(n_chunks, n_acc, OUT_C)
    deg_call = _make_edge_kernel(n_chunks, n_acc, DEG_W)

    h, g = pl.pallas_call(
        _mlp_body,
        grid=_GRID,
        in_specs=[_row_spec(128), _full_spec(128, 128), _full_spec(1, 128),
                  _full_spec(128, OUT_C), _full_spec(1, OUT_C)],
        out_specs=[_row_spec(OUT_C)] * 2,
        out_shape=[_sds((N_NODES, OUT_C))] * 2,
    )(x, W1, b1.reshape(1, -1), W2, b2.reshape(1, -1))

    deg_p = deg_call(src_p, dst_p, ones_tab, zero_blk_deg)
    d0, d1 = deg_p[0, :N_NODES], deg_p[1, :N_NODES]

    zs = pl.pallas_call(
        _prep2_body,
        grid=_GRID,
        in_specs=[_row_spec(OUT_C), _row_spec(DEG_W), _row_spec(DEG_W)],
        out_specs=_row_spec(OUT_C),
        out_shape=_sds((N_NODES, OUT_C)),
    )(h, d0, d1)

    z = h
    for r in range(K_STEPS):
        p_full = edge_call(src_p, dst_p, zs, zero_blk)
        p = p_full[:, :N_NODES]
        if r < K_STEPS - 1:
            z, zs = pl.pallas_call(
                _combine_body,
                grid=_GRID,
                in_specs=[_row_spec(OUT_C)] * 4
                + [_row_spec(DEG_W)] * 2,
                out_specs=[_row_spec(OUT_C)] * 2,
                out_shape=[_sds((N_NODES, OUT_C))] * 2,
            )(p[0], p[1], z, g, d0, d1)
        else:
            out = pl.pallas_call(
                _final_body,
                grid=_GRID,
                in_specs=[_row_spec(OUT_C)] * 4
                + [_row_spec(DEG_W)] * 2,
                out_specs=_row_spec(OUT_C),
                out_shape=_sds((N_NODES, OUT_C)),
            )(p[0], p[1], z, g, d0, d1)
    return out
